# baseline (speedup 1.0000x reference)
import jax
import jax.numpy as jnp
from jax import lax
from jax.experimental import pallas as pl
from jax.experimental.pallas import tpu as pltpu
from jax.experimental.pallas import tpu_sc as plsc

N = 10000
NPAD = 10240
E = 320000
D = 128
H = 256
NG = 64

NC, NS = 2, 16
NW = NC * NS

ECH = 128
NCH_W = 79
EPAD = NW * NCH_W * ECH
NCHUNK = EPAD // ECH

ROWS_PER_SUB = NPAD // NS
GROWS = NPAD // NW
GCH = 80

RB = 512
GRID_D = NPAD // RB


def _vcopy_idx(src3, j, flat):
    for k in range(ECH // 16):
        flat[pl.ds(k * 16, 16)] = src3[j, 0, pl.ds(k * 16, 16)]


def _stage_a_body(v_hbm, dst_hbm, emb_hbm, vemb_out, degp_out,
                  idxg, rowsg, idxd, onesv, zbuf, shdeg, sem):
    c = lax.axis_index("c")
    s = lax.axis_index("s")
    w = c * NS + s

    def _fill(i, _):
        onesv[i] = jnp.ones((16,), jnp.float32)
        zbuf[i] = jnp.zeros((16,), jnp.float32)
        return 0
    lax.fori_loop(0, ECH, _fill, 0)

    for k in range(ROWS_PER_SUB // ECH):
        pltpu.sync_copy(zbuf, shdeg.at[pl.ds(s * ROWS_PER_SUB + k * ECH, ECH)])

    def _g(i, _):
        base = w * GROWS + i * GCH
        pltpu.sync_copy(v_hbm.at[pl.ds(base, GCH)], idxg)
        pltpu.async_copy(emb_hbm.at[idxg], rowsg, sem).wait()
        pltpu.sync_copy(rowsg, vemb_out.at[pl.ds(base, GCH)])
        return 0
    lax.fori_loop(0, GROWS // GCH, _g, 0)

    plsc.subcore_barrier()

    def _d(j, _):
        pltpu.sync_copy(dst_hbm.at[c * (NCH_W * NS) + j * NS + s, 0], idxd)
        pltpu.sync_copy(onesv, shdeg.at[idxd], add=True)
        return 0
    lax.fori_loop(0, NCH_W, _d, 0)

    plsc.subcore_barrier()
    pltpu.sync_copy(shdeg.at[pl.ds(s * ROWS_PER_SUB, ROWS_PER_SUB)],
                    degp_out.at[c, pl.ds(s * ROWS_PER_SUB, ROWS_PER_SUB)])


_stage_a = pl.kernel(
    _stage_a_body,
    out_type=[jax.ShapeDtypeStruct((NPAD, D), jnp.float32),
              jax.ShapeDtypeStruct((NC, NPAD, 16), jnp.float32)],
    mesh=plsc.VectorSubcoreMesh(core_axis_name="c", subcore_axis_name="s"),
    scratch_types=[
        pltpu.VMEM((GCH,), jnp.int32),
        pltpu.VMEM((GCH, D), jnp.float32),
        pltpu.VMEM((ECH,), jnp.int32),
        pltpu.VMEM((ECH, 16), jnp.float32),
        pltpu.VMEM((ECH, 16), jnp.float32),
        pltpu.VMEM_SHARED((NPAD, 16), jnp.float32),
        pltpu.SemaphoreType.DMA,
    ],
)


def _stage_b_body(vemb_ref, w_ref, degp_ref, y_ref, dinv_ref):
    x = vemb_ref[...]
    w = w_ref[...]
    degp = degp_ref[...]
    deg = degp[0, :, 0] + degp[1, :, 0] + 1.0
    dinv = lax.rsqrt(deg)
    xw = jnp.dot(x, w, preferred_element_type=jnp.float32,
                 precision=lax.Precision.HIGHEST)
    y_ref[...] = xw * dinv[:, None]
    dinv_ref[...] = dinv[:, None]


_stage_b = pl.pallas_call(
    _stage_b_body,
    out_shape=[jax.ShapeDtypeStruct((NPAD, D), jnp.float32),
               jax.ShapeDtypeStruct((NPAD, 1), jnp.float32)],
)


def _stage_c_body(src_hbm, dst_hbm, y_hbm, agg_out,
                  idxs, idxd, rows, zbuf, shagg, sem):
    c = lax.axis_index("c")
    s = lax.axis_index("s")
    w = c * NS + s

    def _z(i, _):
        r = i // (D // 16)
        j = i % (D // 16)
        zbuf[r, pl.ds(j * 16, 16)] = jnp.zeros((16,), jnp.float32)
        return 0
    lax.fori_loop(0, ECH * (D // 16), _z, 0)

    for k in range(ROWS_PER_SUB // ECH):
        pltpu.sync_copy(zbuf, shagg.at[pl.ds(s * ROWS_PER_SUB + k * ECH, ECH)])

    plsc.subcore_barrier()

    def _e(i, _):
        ch = c * (NCH_W * NS) + i * NS + s
        pltpu.sync_copy(src_hbm.at[ch, 0], idxs)
        pltpu.sync_copy(dst_hbm.at[ch, 0], idxd)
        pltpu.async_copy(y_hbm.at[idxs], rows, sem).wait()
        pltpu.sync_copy(rows, shagg.at[idxd], add=True)
        return 0
    lax.fori_loop(0, NCH_W, _e, 0)

    plsc.subcore_barrier()
    pltpu.sync_copy(shagg.at[pl.ds(s * ROWS_PER_SUB, ROWS_PER_SUB)],
                    agg_out.at[c, pl.ds(s * ROWS_PER_SUB, ROWS_PER_SUB)])


_stage_c = pl.kernel(
    _stage_c_body,
    out_type=[jax.ShapeDtypeStruct((NC, NPAD, D), jnp.float32)],
    mesh=plsc.VectorSubcoreMesh(core_axis_name="c", subcore_axis_name="s"),
    scratch_types=[
        pltpu.VMEM((ECH,), jnp.int32),
        pltpu.VMEM((ECH,), jnp.int32),
        pltpu.VMEM((ECH, D), jnp.float32),
        pltpu.VMEM((ECH, D), jnp.float32),
        pltpu.VMEM_SHARED((NPAD, D), jnp.float32),
        pltpu.SemaphoreType.DMA,
    ],
)


def _stage_d_body(agg_ref, y_ref, dinv_ref, bi_ref, b_ref, wih_ref, whh_ref,
                  bih_ref, bhh_ref, out_ref, hid_ref, sum_s, max_s, cnt_s):
    step = pl.program_id(0)

    @pl.when(step == 0)
    def _():
        sum_s[...] = jnp.zeros((NG, D), jnp.float32)
        max_s[...] = jnp.full((NG, D), -jnp.inf, jnp.float32)
        cnt_s[...] = jnp.zeros((8, NG), jnp.float32)

    agg = agg_ref[...]
    ctx = dinv_ref[...] * (agg[0] + agg[1] + y_ref[...]) + b_ref[...]
    bi = bi_ref[...][:, 0]
    oh = (bi[:, None] == lax.broadcasted_iota(jnp.int32, (RB, NG), 1)
          ).astype(jnp.float32)
    sum_s[...] = sum_s[...] + jnp.dot(oh.T, ctx,
                                      preferred_element_type=jnp.float32,
                                      precision=lax.Precision.HIGHEST)
    cnt_s[0, :] = cnt_s[0, :] + jnp.sum(oh, axis=0)

    g_lo = bi_ref[0, 0]
    g_hi = jnp.minimum(bi_ref[RB - 1, 0], NG - 1)

    def _mx(gg, _):
        g0 = gg * 8
        gids = lax.broadcasted_iota(jnp.int32, (8, RB), 0) + g0
        msk = bi[None, :] == gids
        v = jnp.max(jnp.where(msk[:, :, None], ctx[None, :, :], -jnp.inf),
                    axis=1)
        max_s[pl.ds(g0, 8), :] = jnp.maximum(max_s[pl.ds(g0, 8), :], v)
        return 0
    lax.fori_loop(0, NG // 8, _mx, 0)

    @pl.when(step == GRID_D - 1)
    def _():
        cnt = jnp.maximum(cnt_s[0, :], 1.0)
        mean = sum_s[...] / cnt[:, None]
        mx = max_s[...]
        wih = wih_ref[...]
        whh = whh_ref[...]
        bih = bih_ref[...]
        bhh = bhh_ref[...]

        def gru_step(xt, h, gh):
            gx = jnp.dot(xt, wih, preferred_element_type=jnp.float32,
                         precision=lax.Precision.HIGHEST) + bih
            r = jax.nn.sigmoid(gx[:, :H] + gh[:, :H])
            z = jax.nn.sigmoid(gx[:, H:2 * H] + gh[:, H:2 * H])
            nn_ = jnp.tanh(gx[:, 2 * H:] + r * gh[:, 2 * H:])
            return (1.0 - z) * nn_ + z * h

        h0 = jnp.zeros((NG, H), jnp.float32)
        gh0 = jnp.broadcast_to(bhh, (NG, 3 * H))
        h1 = gru_step(mean, h0, gh0)
        gh1 = jnp.dot(h1, whh, preferred_element_type=jnp.float32,
                      precision=lax.Precision.HIGHEST) + bhh
        h2 = gru_step(mx, h1, gh1)
        out_ref[0] = h1
        out_ref[1] = h2
        hid_ref[0] = h2


_stage_d = pl.pallas_call(
    _stage_d_body,
    grid=(GRID_D,),
    in_specs=[
        pl.BlockSpec((NC, RB, D), lambda i: (0, i, 0)),
        pl.BlockSpec((RB, D), lambda i: (i, 0)),
        pl.BlockSpec((RB, 1), lambda i: (i, 0)),
        pl.BlockSpec((RB, 1), lambda i: (i, 0)),
        pl.BlockSpec((1, D), lambda i: (0, 0)),
        pl.BlockSpec((D, 3 * H), lambda i: (0, 0)),
        pl.BlockSpec((H, 3 * H), lambda i: (0, 0)),
        pl.BlockSpec((1, 3 * H), lambda i: (0, 0)),
        pl.BlockSpec((1, 3 * H), lambda i: (0, 0)),
    ],
    out_specs=[
        pl.BlockSpec((2, NG, H), lambda i: (0, 0, 0)),
        pl.BlockSpec((1, NG, H), lambda i: (0, 0, 0)),
    ],
    out_shape=[jax.ShapeDtypeStruct((2, NG, H), jnp.float32),
               jax.ShapeDtypeStruct((1, NG, H), jnp.float32)],
    scratch_shapes=[pltpu.VMEM((NG, D), jnp.float32),
                    pltpu.VMEM((NG, D), jnp.float32),
                    pltpu.VMEM((8, NG), jnp.float32)],
)


def kernel(v, e, batch_ind, emb_table, gcn_W, gcn_b, W_ih, W_hh, b_ih, b_hh):
    v_flat = jnp.concatenate([v.reshape(-1).astype(jnp.int32),
                              jnp.zeros((NPAD - N,), jnp.int32)])
    npadedge = EPAD - E
    src = jnp.concatenate([e[0].astype(jnp.int32),
                           jnp.zeros((npadedge,), jnp.int32)]
                          ).reshape(NCHUNK, 1, ECH)
    dst = jnp.concatenate([e[1].astype(jnp.int32),
                           N + (jnp.arange(npadedge, dtype=jnp.int32)
                                % (NPAD - N))]).reshape(NCHUNK, 1, ECH)
    bi = jnp.concatenate([batch_ind.astype(jnp.int32),
                          jnp.full((NPAD - N,), NG, jnp.int32)]).reshape(NPAD, 1)

    vemb, degp = _stage_a(v_flat, dst, emb_table)
    y, dinv = _stage_b(vemb, gcn_W, degp)
    agg, = _stage_c(src, dst, y)
    out, hid = _stage_d(agg, y, dinv, bi, gcn_b.reshape(1, D),
                        W_ih.T, W_hh.T,
                        b_ih.reshape(1, 3 * H), b_hh.reshape(1, 3 * H))
    return out, hid


# speedup vs baseline: 1.1531x; 1.1531x over previous
import jax
import jax.numpy as jnp
from jax import lax
from jax.experimental import pallas as pl
from jax.experimental.pallas import tpu as pltpu
from jax.experimental.pallas import tpu_sc as plsc

N = 10000
NPAD = 10240
E = 320000
D = 128
H = 256
NG = 64

NC, NS = 2, 16
NW = NC * NS

ECH = 128
NCH_W = 79
EPAD = NW * NCH_W * ECH
NCHUNK = EPAD // ECH

ROWS_PER_SUB = NPAD // NS
GROWS = NPAD // NW
GCH = 80

RB = 512
GRID_D = NPAD // RB


def _vcopy_idx(src3, j, flat):
    for k in range(ECH // 16):
        flat[pl.ds(k * 16, 16)] = src3[j, 0, pl.ds(k * 16, 16)]


def _stage_a_body(v_hbm, dst_hbm, emb_hbm, vemb_out, degp_out,
                  idxg, rowsg, idxd, onesv, zbuf, shdeg, sem):
    c = lax.axis_index("c")
    s = lax.axis_index("s")
    w = c * NS + s

    def _fill(i, _):
        onesv[i] = jnp.ones((16,), jnp.float32)
        zbuf[i] = jnp.zeros((16,), jnp.float32)
        return 0
    lax.fori_loop(0, ECH, _fill, 0)

    for k in range(ROWS_PER_SUB // ECH):
        pltpu.sync_copy(zbuf, shdeg.at[pl.ds(s * ROWS_PER_SUB + k * ECH, ECH)])

    def _g(i, _):
        base = w * GROWS + i * GCH
        pltpu.sync_copy(v_hbm.at[pl.ds(base, GCH)], idxg)
        pltpu.async_copy(emb_hbm.at[idxg], rowsg, sem).wait()
        pltpu.sync_copy(rowsg, vemb_out.at[pl.ds(base, GCH)])
        return 0
    lax.fori_loop(0, GROWS // GCH, _g, 0)

    plsc.subcore_barrier()

    def _d(j, _):
        pltpu.sync_copy(
            dst_hbm.at[pl.ds((c * (NCH_W * NS) + j * NS + s) * ECH, ECH)],
            idxd)
        pltpu.sync_copy(onesv, shdeg.at[idxd], add=True)
        return 0
    lax.fori_loop(0, NCH_W, _d, 0)

    plsc.subcore_barrier()
    pltpu.sync_copy(shdeg.at[pl.ds(s * ROWS_PER_SUB, ROWS_PER_SUB)],
                    degp_out.at[c, pl.ds(s * ROWS_PER_SUB, ROWS_PER_SUB)])


_stage_a = pl.kernel(
    _stage_a_body,
    out_type=[jax.ShapeDtypeStruct((NPAD, D), jnp.float32),
              jax.ShapeDtypeStruct((NC, NPAD, 16), jnp.float32)],
    mesh=plsc.VectorSubcoreMesh(core_axis_name="c", subcore_axis_name="s"),
    scratch_types=[
        pltpu.VMEM((GCH,), jnp.int32),
        pltpu.VMEM((GCH, D), jnp.float32),
        pltpu.VMEM((ECH,), jnp.int32),
        pltpu.VMEM((ECH, 16), jnp.float32),
        pltpu.VMEM((ECH, 16), jnp.float32),
        pltpu.VMEM_SHARED((NPAD, 16), jnp.float32),
        pltpu.SemaphoreType.DMA,
    ],
)


def _stage_b_body(vemb_ref, w_ref, degp_ref, y_ref, dinv_ref):
    x = vemb_ref[...]
    w = w_ref[...]
    degp = degp_ref[...]
    deg = degp[0, :, 0] + degp[1, :, 0] + 1.0
    dinv = lax.rsqrt(deg)
    xw = jnp.dot(x, w, preferred_element_type=jnp.float32,
                 precision=lax.Precision.HIGHEST)
    y_ref[...] = xw * dinv[:, None]
    dinv_ref[...] = dinv[:, None]


_stage_b = pl.pallas_call(
    _stage_b_body,
    out_shape=[jax.ShapeDtypeStruct((NPAD, D), jnp.float32),
               jax.ShapeDtypeStruct((NPAD, 1), jnp.float32)],
)


def _stage_c_body(src_hbm, dst_hbm, y_hbm, agg_out,
                  idxs, idxd, rows, zbuf, shagg, sem):
    c = lax.axis_index("c")
    s = lax.axis_index("s")
    w = c * NS + s

    def _z(i, _):
        r = i // (D // 16)
        j = i % (D // 16)
        zbuf[r, pl.ds(j * 16, 16)] = jnp.zeros((16,), jnp.float32)
        return 0
    lax.fori_loop(0, ECH * (D // 16), _z, 0)

    for k in range(ROWS_PER_SUB // ECH):
        pltpu.sync_copy(zbuf, shagg.at[pl.ds(s * ROWS_PER_SUB + k * ECH, ECH)])

    plsc.subcore_barrier()

    def _e(i, _):
        base = (c * (NCH_W * NS) + i * NS + s) * ECH
        pltpu.sync_copy(src_hbm.at[pl.ds(base, ECH)], idxs)
        pltpu.sync_copy(dst_hbm.at[pl.ds(base, ECH)], idxd)
        pltpu.async_copy(y_hbm.at[idxs], rows, sem).wait()
        pltpu.sync_copy(rows, shagg.at[idxd], add=True)
        return 0
    lax.fori_loop(0, NCH_W, _e, 0)

    plsc.subcore_barrier()
    pltpu.sync_copy(shagg.at[pl.ds(s * ROWS_PER_SUB, ROWS_PER_SUB)],
                    agg_out.at[c, pl.ds(s * ROWS_PER_SUB, ROWS_PER_SUB)])


_stage_c = pl.kernel(
    _stage_c_body,
    out_type=[jax.ShapeDtypeStruct((NC, NPAD, D), jnp.float32)],
    mesh=plsc.VectorSubcoreMesh(core_axis_name="c", subcore_axis_name="s"),
    scratch_types=[
        pltpu.VMEM((ECH,), jnp.int32),
        pltpu.VMEM((ECH,), jnp.int32),
        pltpu.VMEM((ECH, D), jnp.float32),
        pltpu.VMEM((ECH, D), jnp.float32),
        pltpu.VMEM_SHARED((NPAD, D), jnp.float32),
        pltpu.SemaphoreType.DMA,
    ],
)


def _stage_d_body(agg_ref, y_ref, dinv_ref, bi_ref, b_ref, wih_ref, whh_ref,
                  bih_ref, bhh_ref, out_ref, hid_ref, sum_s, max_s, cnt_s):
    step = pl.program_id(0)

    @pl.when(step == 0)
    def _():
        sum_s[...] = jnp.zeros((NG, D), jnp.float32)
        max_s[...] = jnp.full((NG, D), -jnp.inf, jnp.float32)
        cnt_s[...] = jnp.zeros((8, NG), jnp.float32)

    agg = agg_ref[...]
    ctx = dinv_ref[...] * (agg[0] + agg[1] + y_ref[...]) + b_ref[...]
    bi = bi_ref[...][:, 0]
    oh = (bi[:, None] == lax.broadcasted_iota(jnp.int32, (RB, NG), 1)
          ).astype(jnp.float32)
    sum_s[...] = sum_s[...] + jnp.dot(oh.T, ctx,
                                      preferred_element_type=jnp.float32,
                                      precision=lax.Precision.HIGHEST)
    cnt_s[0, :] = cnt_s[0, :] + jnp.sum(oh, axis=0)

    g_lo = bi_ref[0, 0]
    g_hi = jnp.minimum(bi_ref[RB - 1, 0], NG - 1)

    def _mx(gg, _):
        g0 = gg * 8
        gids = lax.broadcasted_iota(jnp.int32, (8, RB), 0) + g0
        msk = bi[None, :] == gids
        v = jnp.max(jnp.where(msk[:, :, None], ctx[None, :, :], -jnp.inf),
                    axis=1)
        max_s[pl.ds(g0, 8), :] = jnp.maximum(max_s[pl.ds(g0, 8), :], v)
        return 0
    lax.fori_loop(g_lo // 8, g_hi // 8 + 1, _mx, 0)

    @pl.when(step == GRID_D - 1)
    def _():
        cnt = jnp.maximum(cnt_s[0, :], 1.0)
        mean = sum_s[...] / cnt[:, None]
        mx = max_s[...]
        wih = wih_ref[...]
        whh = whh_ref[...]
        bih = bih_ref[...]
        bhh = bhh_ref[...]

        def gru_step(xt, h, gh):
            gx = jnp.dot(xt, wih, preferred_element_type=jnp.float32,
                         precision=lax.Precision.HIGHEST) + bih
            r = jax.nn.sigmoid(gx[:, :H] + gh[:, :H])
            z = jax.nn.sigmoid(gx[:, H:2 * H] + gh[:, H:2 * H])
            nn_ = jnp.tanh(gx[:, 2 * H:] + r * gh[:, 2 * H:])
            return (1.0 - z) * nn_ + z * h

        h0 = jnp.zeros((NG, H), jnp.float32)
        gh0 = jnp.broadcast_to(bhh, (NG, 3 * H))
        h1 = gru_step(mean, h0, gh0)
        gh1 = jnp.dot(h1, whh, preferred_element_type=jnp.float32,
                      precision=lax.Precision.HIGHEST) + bhh
        h2 = gru_step(mx, h1, gh1)
        out_ref[0] = h1
        out_ref[1] = h2
        hid_ref[0] = h2


_stage_d = pl.pallas_call(
    _stage_d_body,
    grid=(GRID_D,),
    in_specs=[
        pl.BlockSpec((NC, RB, D), lambda i: (0, i, 0)),
        pl.BlockSpec((RB, D), lambda i: (i, 0)),
        pl.BlockSpec((RB, 1), lambda i: (i, 0)),
        pl.BlockSpec((RB, 1), lambda i: (i, 0)),
        pl.BlockSpec((1, D), lambda i: (0, 0)),
        pl.BlockSpec((D, 3 * H), lambda i: (0, 0)),
        pl.BlockSpec((H, 3 * H), lambda i: (0, 0)),
        pl.BlockSpec((1, 3 * H), lambda i: (0, 0)),
        pl.BlockSpec((1, 3 * H), lambda i: (0, 0)),
    ],
    out_specs=[
        pl.BlockSpec((2, NG, H), lambda i: (0, 0, 0)),
        pl.BlockSpec((1, NG, H), lambda i: (0, 0, 0)),
    ],
    out_shape=[jax.ShapeDtypeStruct((2, NG, H), jnp.float32),
               jax.ShapeDtypeStruct((1, NG, H), jnp.float32)],
    scratch_shapes=[pltpu.VMEM((NG, D), jnp.float32),
                    pltpu.VMEM((NG, D), jnp.float32),
                    pltpu.VMEM((8, NG), jnp.float32)],
)


def kernel(v, e, batch_ind, emb_table, gcn_W, gcn_b, W_ih, W_hh, b_ih, b_hh):
    v_flat = jnp.concatenate([v.reshape(-1).astype(jnp.int32),
                              jnp.zeros((NPAD - N,), jnp.int32)])
    npadedge = EPAD - E
    src = jnp.concatenate([e[0].astype(jnp.int32),
                           jnp.zeros((npadedge,), jnp.int32)]
                          )
    dst = jnp.concatenate([e[1].astype(jnp.int32),
                           N + (jnp.arange(npadedge, dtype=jnp.int32)
                                % (NPAD - N))])
    bi = jnp.concatenate([batch_ind.astype(jnp.int32),
                          jnp.full((NPAD - N,), NG, jnp.int32)]).reshape(NPAD, 1)

    vemb, degp = _stage_a(v_flat, dst, emb_table)
    y, dinv = _stage_b(vemb, gcn_W, degp)
    agg, = _stage_c(src, dst, y)
    out, hid = _stage_d(agg, y, dinv, bi, gcn_b.reshape(1, D),
                        W_ih.T, W_hh.T,
                        b_ih.reshape(1, 3 * H), b_hh.reshape(1, 3 * H))
    return out, hid


# speedup vs baseline: 1.4991x; 1.3000x over previous
import jax
import jax.numpy as jnp
from jax import lax
from jax.experimental import pallas as pl
from jax.experimental.pallas import tpu as pltpu
from jax.experimental.pallas import tpu_sc as plsc

N = 10000
NPAD = 10240
E = 320000
D = 128
H = 256
NG = 64

NC, NS = 2, 16
NW = NC * NS

ECH = 128
NCH_W = 79
EPAD = NW * NCH_W * ECH
NCHUNK = EPAD // ECH

ROWS_PER_SUB = NPAD // NS
GROWS = NPAD // NW
GCH = 80

RB = 512
GRID_D = NPAD // RB


def _vcopy_idx(src3, j, flat):
    for k in range(ECH // 16):
        flat[pl.ds(k * 16, 16)] = src3[j, 0, pl.ds(k * 16, 16)]


def _stage_a_body(v_hbm, dst_hbm, emb_hbm, vemb_out, degp_out,
                  idxg, rowsg, idxd, onesv, zbuf, shdeg, sem):
    c = lax.axis_index("c")
    s = lax.axis_index("s")
    w = c * NS + s

    def _fill(i, _):
        onesv[i] = jnp.ones((16,), jnp.float32)
        zbuf[i] = jnp.zeros((16,), jnp.float32)
        return 0
    lax.fori_loop(0, ECH, _fill, 0)

    for k in range(ROWS_PER_SUB // ECH):
        pltpu.sync_copy(zbuf, shdeg.at[pl.ds(s * ROWS_PER_SUB + k * ECH, ECH)])

    def _g(i, _):
        base = w * GROWS + i * GCH
        pltpu.sync_copy(v_hbm.at[pl.ds(base, GCH)], idxg)
        pltpu.async_copy(emb_hbm.at[idxg], rowsg, sem).wait()
        pltpu.sync_copy(rowsg, vemb_out.at[pl.ds(base, GCH)])
        return 0
    lax.fori_loop(0, GROWS // GCH, _g, 0)

    plsc.subcore_barrier()

    def _d(j, _):
        ch = c * (NCH_W * NS) + j * NS + s
        pltpu.sync_copy(dst_hbm.at[pl.ds(ch * ECH, ECH)], idxd)

        @pl.when(ch < E // ECH)
        def _():
            pltpu.sync_copy(onesv, shdeg.at[idxd], add=True)

        @pl.when(ch >= E // ECH)
        def _():
            pltpu.sync_copy(zbuf, shdeg.at[idxd], add=True)
        return 0
    lax.fori_loop(0, NCH_W, _d, 0)

    plsc.subcore_barrier()
    pltpu.sync_copy(shdeg.at[pl.ds(s * ROWS_PER_SUB, ROWS_PER_SUB)],
                    degp_out.at[c, pl.ds(s * ROWS_PER_SUB, ROWS_PER_SUB)])


_stage_a = pl.kernel(
    _stage_a_body,
    out_type=[jax.ShapeDtypeStruct((NPAD, D), jnp.float32),
              jax.ShapeDtypeStruct((NC, NPAD, 16), jnp.float32)],
    mesh=plsc.VectorSubcoreMesh(core_axis_name="c", subcore_axis_name="s"),
    scratch_types=[
        pltpu.VMEM((GCH,), jnp.int32),
        pltpu.VMEM((GCH, D), jnp.float32),
        pltpu.VMEM((ECH,), jnp.int32),
        pltpu.VMEM((ECH, 16), jnp.float32),
        pltpu.VMEM((ECH, 16), jnp.float32),
        pltpu.VMEM_SHARED((NPAD, 16), jnp.float32),
        pltpu.SemaphoreType.DMA,
    ],
)


def _stage_b_body(vemb_ref, w_ref, degp_ref, y_ref, dinv_ref):
    x = vemb_ref[...]
    w = w_ref[...]
    degp = degp_ref[...]
    deg = degp[0, :, 0] + degp[1, :, 0] + 1.0
    dinv = lax.rsqrt(deg)
    xw = jnp.dot(x, w, preferred_element_type=jnp.float32,
                 precision=lax.Precision.HIGHEST)
    rows = lax.broadcasted_iota(jnp.int32, (NPAD, 1), 0)
    y_ref[...] = jnp.where(rows < N, xw * dinv[:, None], 0.0)
    dinv_ref[...] = dinv[:, None]


_stage_b = pl.pallas_call(
    _stage_b_body,
    out_shape=[jax.ShapeDtypeStruct((NPAD, D), jnp.float32),
               jax.ShapeDtypeStruct((NPAD, 1), jnp.float32)],
)


def _stage_c_body(src_hbm, dst_hbm, y_hbm, agg_out,
                  idxs, idxd, rows, zbuf, shagg, sem):
    c = lax.axis_index("c")
    s = lax.axis_index("s")
    w = c * NS + s

    def _z(i, _):
        r = i // (D // 16)
        j = i % (D // 16)
        zbuf[r, pl.ds(j * 16, 16)] = jnp.zeros((16,), jnp.float32)
        return 0
    lax.fori_loop(0, ECH * (D // 16), _z, 0)

    for k in range(ROWS_PER_SUB // ECH):
        pltpu.sync_copy(zbuf, shagg.at[pl.ds(s * ROWS_PER_SUB + k * ECH, ECH)])

    plsc.subcore_barrier()

    def _e(i, _):
        base = (c * (NCH_W * NS) + i * NS + s) * ECH
        pltpu.sync_copy(src_hbm.at[pl.ds(base, ECH)], idxs)
        pltpu.sync_copy(dst_hbm.at[pl.ds(base, ECH)], idxd)
        pltpu.async_copy(y_hbm.at[idxs], rows, sem).wait()
        pltpu.sync_copy(rows, shagg.at[idxd], add=True)
        return 0
    lax.fori_loop(0, NCH_W, _e, 0)

    plsc.subcore_barrier()
    pltpu.sync_copy(shagg.at[pl.ds(s * ROWS_PER_SUB, ROWS_PER_SUB)],
                    agg_out.at[c, pl.ds(s * ROWS_PER_SUB, ROWS_PER_SUB)])


_stage_c = pl.kernel(
    _stage_c_body,
    out_type=[jax.ShapeDtypeStruct((NC, NPAD, D), jnp.float32)],
    mesh=plsc.VectorSubcoreMesh(core_axis_name="c", subcore_axis_name="s"),
    scratch_types=[
        pltpu.VMEM((ECH,), jnp.int32),
        pltpu.VMEM((ECH,), jnp.int32),
        pltpu.VMEM((ECH, D), jnp.float32),
        pltpu.VMEM((ECH, D), jnp.float32),
        pltpu.VMEM_SHARED((NPAD, D), jnp.float32),
        pltpu.SemaphoreType.DMA,
    ],
)


def _stage_d_body(agg_ref, y_ref, dinv_ref, bi_ref, b_ref, wih_ref, whh_ref,
                  bih_ref, bhh_ref, out_ref, hid_ref, sum_s, max_s, cnt_s):
    step = pl.program_id(0)

    @pl.when(step == 0)
    def _():
        sum_s[...] = jnp.zeros((NG, D), jnp.float32)
        max_s[...] = jnp.full((NG, D), -jnp.inf, jnp.float32)
        cnt_s[...] = jnp.zeros((8, NG), jnp.float32)

    agg = agg_ref[...]
    ctx = dinv_ref[...] * (agg[0] + agg[1] + y_ref[...]) + b_ref[...]
    bi = bi_ref[...][:, 0]
    oh = (bi[:, None] == lax.broadcasted_iota(jnp.int32, (RB, NG), 1)
          ).astype(jnp.float32)
    sum_s[...] = sum_s[...] + jnp.dot(oh.T, ctx,
                                      preferred_element_type=jnp.float32,
                                      precision=lax.Precision.HIGHEST)
    cnt_s[0, :] = cnt_s[0, :] + jnp.sum(oh, axis=0)

    g_lo = bi_ref[0, 0]
    g_hi = jnp.minimum(bi_ref[RB - 1, 0], NG - 1)

    def _mx(gg, _):
        g0 = gg * 8
        gids = lax.broadcasted_iota(jnp.int32, (8, RB), 0) + g0
        msk = bi[None, :] == gids
        v = jnp.max(jnp.where(msk[:, :, None], ctx[None, :, :], -jnp.inf),
                    axis=1)
        max_s[pl.ds(g0, 8), :] = jnp.maximum(max_s[pl.ds(g0, 8), :], v)
        return 0
    lax.fori_loop(g_lo // 8, g_hi // 8 + 1, _mx, 0)

    @pl.when(step == GRID_D - 1)
    def _():
        cnt = jnp.maximum(cnt_s[0, :], 1.0)
        mean = sum_s[...] / cnt[:, None]
        mx = max_s[...]
        wih = wih_ref[...]
        whh = whh_ref[...]
        bih = bih_ref[...]
        bhh = bhh_ref[...]

        def gru_step(xt, h, gh):
            gx = jnp.dot(xt, wih, preferred_element_type=jnp.float32,
                         precision=lax.Precision.HIGHEST) + bih
            r = jax.nn.sigmoid(gx[:, :H] + gh[:, :H])
            z = jax.nn.sigmoid(gx[:, H:2 * H] + gh[:, H:2 * H])
            nn_ = jnp.tanh(gx[:, 2 * H:] + r * gh[:, 2 * H:])
            return (1.0 - z) * nn_ + z * h

        h0 = jnp.zeros((NG, H), jnp.float32)
        gh0 = jnp.broadcast_to(bhh, (NG, 3 * H))
        h1 = gru_step(mean, h0, gh0)
        gh1 = jnp.dot(h1, whh, preferred_element_type=jnp.float32,
                      precision=lax.Precision.HIGHEST) + bhh
        h2 = gru_step(mx, h1, gh1)
        out_ref[0] = h1
        out_ref[1] = h2
        hid_ref[0] = h2


_stage_d = pl.pallas_call(
    _stage_d_body,
    grid=(GRID_D,),
    in_specs=[
        pl.BlockSpec((NC, RB, D), lambda i: (0, i, 0)),
        pl.BlockSpec((RB, D), lambda i: (i, 0)),
        pl.BlockSpec((RB, 1), lambda i: (i, 0)),
        pl.BlockSpec((RB, 1), lambda i: (i, 0)),
        pl.BlockSpec((1, D), lambda i: (0, 0)),
        pl.BlockSpec((D, 3 * H), lambda i: (0, 0)),
        pl.BlockSpec((H, 3 * H), lambda i: (0, 0)),
        pl.BlockSpec((1, 3 * H), lambda i: (0, 0)),
        pl.BlockSpec((1, 3 * H), lambda i: (0, 0)),
    ],
    out_specs=[
        pl.BlockSpec((2, NG, H), lambda i: (0, 0, 0)),
        pl.BlockSpec((1, NG, H), lambda i: (0, 0, 0)),
    ],
    out_shape=[jax.ShapeDtypeStruct((2, NG, H), jnp.float32),
               jax.ShapeDtypeStruct((1, NG, H), jnp.float32)],
    scratch_shapes=[pltpu.VMEM((NG, D), jnp.float32),
                    pltpu.VMEM((NG, D), jnp.float32),
                    pltpu.VMEM((8, NG), jnp.float32)],
)


def kernel(v, e, batch_ind, emb_table, gcn_W, gcn_b, W_ih, W_hh, b_ih, b_hh):
    v_flat = jnp.concatenate([v.reshape(-1).astype(jnp.int32),
                              jnp.zeros((NPAD - N,), jnp.int32)])
    npadedge = EPAD - E
    pr = jnp.arange(npadedge, dtype=jnp.int32)
    src = jnp.concatenate([e[0].astype(jnp.int32), N + pr % (NPAD - N)])
    dst = jnp.concatenate([e[1].astype(jnp.int32), (pr * 37) % N])
    bi = jnp.concatenate([batch_ind.astype(jnp.int32),
                          jnp.full((NPAD - N,), NG, jnp.int32)]).reshape(NPAD, 1)

    vemb, degp = _stage_a(v_flat, dst, emb_table)
    y, dinv = _stage_b(vemb, gcn_W, degp)
    agg, = _stage_c(src, dst, y)
    out, hid = _stage_d(agg, y, dinv, bi, gcn_b.reshape(1, D),
                        W_ih.T, W_hh.T,
                        b_ih.reshape(1, 3 * H), b_hh.reshape(1, 3 * H))
    return out, hid


# speedup vs baseline: 2.1146x; 1.4106x over previous
import jax
import jax.numpy as jnp
from jax import lax
from jax.experimental import pallas as pl
from jax.experimental.pallas import tpu as pltpu
from jax.experimental.pallas import tpu_sc as plsc

N = 10000
NPAD = 10240
E = 320000
D = 128
H = 256
NG = 64

NC, NS = 2, 16
NW = NC * NS

ECH = 128
NCH_W = 79
EPAD = NW * NCH_W * ECH
NCHUNK = EPAD // ECH

ROWS_PER_SUB = NPAD // NS
GROWS = NPAD // NW
GCH = 80

RB = 512
GRID_D = NPAD // RB


def _vcopy_idx(src3, j, flat):
    for k in range(ECH // 16):
        flat[pl.ds(k * 16, 16)] = src3[j, 0, pl.ds(k * 16, 16)]


def _stage_a_body(v_hbm, dst_hbm, emb_hbm, vemb_out, degp_out,
                  idxg, rowsg, idxd, onesv, zbuf, shdeg, sem):
    c = lax.axis_index("c")
    s = lax.axis_index("s")
    w = c * NS + s

    def _fill(i, _):
        onesv[i] = jnp.ones((16,), jnp.float32)
        zbuf[i] = jnp.zeros((16,), jnp.float32)
        return 0
    lax.fori_loop(0, ECH, _fill, 0)

    for k in range(ROWS_PER_SUB // ECH):
        pltpu.sync_copy(zbuf, shdeg.at[pl.ds(s * ROWS_PER_SUB + k * ECH, ECH)])

    def _g(i, _):
        base = w * GROWS + i * GCH
        pltpu.sync_copy(v_hbm.at[pl.ds(base, GCH)], idxg)
        pltpu.async_copy(emb_hbm.at[idxg], rowsg, sem).wait()
        pltpu.sync_copy(rowsg, vemb_out.at[pl.ds(base, GCH)])
        return 0
    lax.fori_loop(0, GROWS // GCH, _g, 0)

    plsc.subcore_barrier()

    def _d(j, _):
        ch = c * (NCH_W * NS) + j * NS + s
        pltpu.sync_copy(dst_hbm.at[pl.ds(ch * ECH, ECH)], idxd)

        @pl.when(ch < E // ECH)
        def _():
            pltpu.sync_copy(onesv, shdeg.at[idxd], add=True)

        @pl.when(ch >= E // ECH)
        def _():
            pltpu.sync_copy(zbuf, shdeg.at[idxd], add=True)
        return 0
    lax.fori_loop(0, NCH_W, _d, 0)

    plsc.subcore_barrier()
    pltpu.sync_copy(shdeg.at[pl.ds(s * ROWS_PER_SUB, ROWS_PER_SUB)],
                    degp_out.at[c, pl.ds(s * ROWS_PER_SUB, ROWS_PER_SUB)])


_stage_a = pl.kernel(
    _stage_a_body,
    out_type=[jax.ShapeDtypeStruct((NPAD, D), jnp.float32),
              jax.ShapeDtypeStruct((NC, NPAD, 16), jnp.float32)],
    mesh=plsc.VectorSubcoreMesh(core_axis_name="c", subcore_axis_name="s"),
    scratch_types=[
        pltpu.VMEM((GCH,), jnp.int32),
        pltpu.VMEM((GCH, D), jnp.float32),
        pltpu.VMEM((ECH,), jnp.int32),
        pltpu.VMEM((ECH, 16), jnp.float32),
        pltpu.VMEM((ECH, 16), jnp.float32),
        pltpu.VMEM_SHARED((NPAD, 16), jnp.float32),
        pltpu.SemaphoreType.DMA,
    ],
)


def _stage_b_body(vemb_ref, w_ref, degp_ref, y_ref, dinv_ref):
    x = vemb_ref[...]
    w = w_ref[...]
    degp = degp_ref[...]
    deg = degp[0, :, 0] + degp[1, :, 0] + 1.0
    dinv = lax.rsqrt(deg)
    xw = jnp.dot(x, w, preferred_element_type=jnp.float32,
                 precision=lax.Precision.HIGHEST)
    rows = lax.broadcasted_iota(jnp.int32, (NPAD, 1), 0)
    y_ref[...] = jnp.where(rows < N, xw * dinv[:, None], 0.0)
    dinv_ref[...] = dinv[:, None]


_stage_b = pl.pallas_call(
    _stage_b_body,
    out_shape=[jax.ShapeDtypeStruct((NPAD, D), jnp.float32),
               jax.ShapeDtypeStruct((NPAD, 1), jnp.float32)],
)


def _stage_c_body(src_hbm, dst_hbm, y_hbm, agg_out,
                  idxs0, idxd0, idxs1, idxd1, rows0, rows1, shagg,
                  sem0, sem1, semi0, semi1):
    c = lax.axis_index("c")
    s = lax.axis_index("s")

    def _z(i, _):
        r = i // (D // 16)
        j = i % (D // 16)
        rows0[r, pl.ds(j * 16, 16)] = jnp.zeros((16,), jnp.float32)
        return 0
    lax.fori_loop(0, ECH * (D // 16), _z, 0)

    for k in range(ROWS_PER_SUB // ECH):
        pltpu.sync_copy(rows0, shagg.at[pl.ds(s * ROWS_PER_SUB + k * ECH, ECH)])

    plsc.subcore_barrier()

    def _ch(k):
        return (c * (NCH_W * NS) + k * NS + s) * ECH

    def _fire_idx(k, bs, bd, sem):
        pltpu.async_copy(src_hbm.at[pl.ds(_ch(k), ECH)], bs, sem)
        pltpu.async_copy(dst_hbm.at[pl.ds(_ch(k), ECH)], bd, sem)

    def _wait_idx(k, bs, bd, sem):
        pltpu.make_async_copy(src_hbm.at[pl.ds(_ch(k), ECH)], bs, sem).wait()
        pltpu.make_async_copy(dst_hbm.at[pl.ds(_ch(k), ECH)], bd, sem).wait()

    _fire_idx(0, idxs0, idxd0, semi0)
    _wait_idx(0, idxs0, idxd0, semi0)
    pltpu.async_copy(y_hbm.at[idxs0], rows0, sem0)
    _fire_idx(1, idxs1, idxd1, semi1)

    def _e(i, _):
        a = 2 * i
        pltpu.make_async_copy(y_hbm.at[idxs0], rows0, sem0).wait()
        _wait_idx(a + 1, idxs1, idxd1, semi1)
        pltpu.async_copy(y_hbm.at[idxs1], rows1, sem1)
        pltpu.sync_copy(rows0, shagg.at[idxd0], add=True)
        _fire_idx(a + 2, idxs0, idxd0, semi0)

        pltpu.make_async_copy(y_hbm.at[idxs1], rows1, sem1).wait()
        _wait_idx(a + 2, idxs0, idxd0, semi0)
        pltpu.async_copy(y_hbm.at[idxs0], rows0, sem0)
        pltpu.sync_copy(rows1, shagg.at[idxd1], add=True)

        @pl.when(a + 3 < NCH_W)
        def _():
            _fire_idx(a + 3, idxs1, idxd1, semi1)
        return 0
    lax.fori_loop(0, (NCH_W - 1) // 2, _e, 0)

    pltpu.make_async_copy(y_hbm.at[idxs0], rows0, sem0).wait()
    pltpu.sync_copy(rows0, shagg.at[idxd0], add=True)

    plsc.subcore_barrier()
    pltpu.sync_copy(shagg.at[pl.ds(s * ROWS_PER_SUB, ROWS_PER_SUB)],
                    agg_out.at[c, pl.ds(s * ROWS_PER_SUB, ROWS_PER_SUB)])


_stage_c = pl.kernel(
    _stage_c_body,
    out_type=[jax.ShapeDtypeStruct((NC, NPAD, D), jnp.float32)],
    mesh=plsc.VectorSubcoreMesh(core_axis_name="c", subcore_axis_name="s"),
    scratch_types=[
        pltpu.VMEM((ECH,), jnp.int32),
        pltpu.VMEM((ECH,), jnp.int32),
        pltpu.VMEM((ECH,), jnp.int32),
        pltpu.VMEM((ECH,), jnp.int32),
        pltpu.VMEM((ECH, D), jnp.float32),
        pltpu.VMEM((ECH, D), jnp.float32),
        pltpu.VMEM_SHARED((NPAD, D), jnp.float32),
        pltpu.SemaphoreType.DMA,
        pltpu.SemaphoreType.DMA,
        pltpu.SemaphoreType.DMA,
        pltpu.SemaphoreType.DMA,
    ],
)


def _stage_d_body(agg_ref, y_ref, dinv_ref, bi_ref, b_ref, wih_ref, whh_ref,
                  bih_ref, bhh_ref, out_ref, hid_ref, sum_s, max_s, cnt_s):
    step = pl.program_id(0)

    @pl.when(step == 0)
    def _():
        sum_s[...] = jnp.zeros((NG, D), jnp.float32)
        max_s[...] = jnp.full((NG, D), -jnp.inf, jnp.float32)
        cnt_s[...] = jnp.zeros((8, NG), jnp.float32)

    agg = agg_ref[...]
    ctx = dinv_ref[...] * (agg[0] + agg[1] + y_ref[...]) + b_ref[...]
    bi = bi_ref[...][:, 0]
    oh = (bi[:, None] == lax.broadcasted_iota(jnp.int32, (RB, NG), 1)
          ).astype(jnp.float32)
    sum_s[...] = sum_s[...] + jnp.dot(oh.T, ctx,
                                      preferred_element_type=jnp.float32,
                                      precision=lax.Precision.HIGHEST)
    cnt_s[0, :] = cnt_s[0, :] + jnp.sum(oh, axis=0)

    g_lo = bi_ref[0, 0]
    g_hi = jnp.minimum(bi_ref[RB - 1, 0], NG - 1)

    def _mx(gg, _):
        g0 = gg * 8
        gids = lax.broadcasted_iota(jnp.int32, (8, RB), 0) + g0
        msk = bi[None, :] == gids
        v = jnp.max(jnp.where(msk[:, :, None], ctx[None, :, :], -jnp.inf),
                    axis=1)
        max_s[pl.ds(g0, 8), :] = jnp.maximum(max_s[pl.ds(g0, 8), :], v)
        return 0
    lax.fori_loop(g_lo // 8, g_hi // 8 + 1, _mx, 0)

    @pl.when(step == GRID_D - 1)
    def _():
        cnt = jnp.maximum(cnt_s[0, :], 1.0)
        mean = sum_s[...] / cnt[:, None]
        mx = max_s[...]
        wih = wih_ref[...]
        whh = whh_ref[...]
        bih = bih_ref[...]
        bhh = bhh_ref[...]

        def gru_step(xt, h, gh):
            gx = jnp.dot(xt, wih, preferred_element_type=jnp.float32,
                         precision=lax.Precision.HIGHEST) + bih
            r = jax.nn.sigmoid(gx[:, :H] + gh[:, :H])
            z = jax.nn.sigmoid(gx[:, H:2 * H] + gh[:, H:2 * H])
            nn_ = jnp.tanh(gx[:, 2 * H:] + r * gh[:, 2 * H:])
            return (1.0 - z) * nn_ + z * h

        h0 = jnp.zeros((NG, H), jnp.float32)
        gh0 = jnp.broadcast_to(bhh, (NG, 3 * H))
        h1 = gru_step(mean, h0, gh0)
        gh1 = jnp.dot(h1, whh, preferred_element_type=jnp.float32,
                      precision=lax.Precision.HIGHEST) + bhh
        h2 = gru_step(mx, h1, gh1)
        out_ref[0] = h1
        out_ref[1] = h2
        hid_ref[0] = h2


_stage_d = pl.pallas_call(
    _stage_d_body,
    grid=(GRID_D,),
    in_specs=[
        pl.BlockSpec((NC, RB, D), lambda i: (0, i, 0)),
        pl.BlockSpec((RB, D), lambda i: (i, 0)),
        pl.BlockSpec((RB, 1), lambda i: (i, 0)),
        pl.BlockSpec((RB, 1), lambda i: (i, 0)),
        pl.BlockSpec((1, D), lambda i: (0, 0)),
        pl.BlockSpec((D, 3 * H), lambda i: (0, 0)),
        pl.BlockSpec((H, 3 * H), lambda i: (0, 0)),
        pl.BlockSpec((1, 3 * H), lambda i: (0, 0)),
        pl.BlockSpec((1, 3 * H), lambda i: (0, 0)),
    ],
    out_specs=[
        pl.BlockSpec((2, NG, H), lambda i: (0, 0, 0)),
        pl.BlockSpec((1, NG, H), lambda i: (0, 0, 0)),
    ],
    out_shape=[jax.ShapeDtypeStruct((2, NG, H), jnp.float32),
               jax.ShapeDtypeStruct((1, NG, H), jnp.float32)],
    scratch_shapes=[pltpu.VMEM((NG, D), jnp.float32),
                    pltpu.VMEM((NG, D), jnp.float32),
                    pltpu.VMEM((8, NG), jnp.float32)],
)


def kernel(v, e, batch_ind, emb_table, gcn_W, gcn_b, W_ih, W_hh, b_ih, b_hh):
    v_flat = jnp.concatenate([v.reshape(-1).astype(jnp.int32),
                              jnp.zeros((NPAD - N,), jnp.int32)])
    npadedge = EPAD - E
    pr = jnp.arange(npadedge, dtype=jnp.int32)
    src = jnp.concatenate([e[0].astype(jnp.int32), N + pr % (NPAD - N)])
    dst = jnp.concatenate([e[1].astype(jnp.int32), (pr * 37) % N])
    bi = jnp.concatenate([batch_ind.astype(jnp.int32),
                          jnp.full((NPAD - N,), NG, jnp.int32)]).reshape(NPAD, 1)

    vemb, degp = _stage_a(v_flat, dst, emb_table)
    y, dinv = _stage_b(vemb, gcn_W, degp)
    agg, = _stage_c(src, dst, y)
    out, hid = _stage_d(agg, y, dinv, bi, gcn_b.reshape(1, D),
                        W_ih.T, W_hh.T,
                        b_ih.reshape(1, 3 * H), b_hh.reshape(1, 3 * H))
    return out, hid
